# Initial kernel scaffold; baseline (speedup 1.0000x reference)
import jax
import jax.numpy as jnp
from jax.experimental import pallas as pl


def kernel(text, W, b):
    raise NotImplementedError("write your pallas kernel here")


# speedup vs baseline: 24.4064x; 24.4064x over previous
import functools

import jax
import jax.numpy as jnp
from jax import lax
from jax.experimental import pallas as pl
from jax.experimental.pallas import tpu as pltpu
from jax.experimental.pallas import tpu_sc as plsc

_NUM_CORES = 2
_NUM_SUBCORES = 16
_NUM_WORKERS = _NUM_CORES * _NUM_SUBCORES
_LANES = 16


@functools.lru_cache(maxsize=None)
def _make_gather_sum(L, B, V):
    b_per_w = B // _NUM_WORKERS
    groups = b_per_w // _LANES
    mesh = plsc.VectorSubcoreMesh(core_axis_name="c", subcore_axis_name="s")

    @functools.partial(
        pl.kernel,
        mesh=mesh,
        out_type=jax.ShapeDtypeStruct((B,), jnp.float32),
        scratch_types=[
            pltpu.VMEM((V,), jnp.float32),
            pltpu.VMEM((L, b_per_w), jnp.int32),
            pltpu.VMEM((b_per_w,), jnp.float32),
        ],
        compiler_params=pltpu.CompilerParams(needs_layout_passes=False),
    )
    def gather_sum(w_hbm, tex_hbm, out_hbm, w_v, tex_v, out_v):
        wid = lax.axis_index("s") * _NUM_CORES + lax.axis_index("c")
        pltpu.sync_copy(w_hbm, w_v)
        pltpu.sync_copy(tex_hbm.at[wid], tex_v)

        def body(l, accs):
            out = []
            for g in range(groups):
                idx = tex_v[l, pl.ds(g * _LANES, _LANES)]
                vals = plsc.load_gather(w_v, [idx])
                out.append(accs[g] + vals)
            return tuple(out)

        init = tuple(jnp.zeros((_LANES,), jnp.float32) for _ in range(groups))
        accs = lax.fori_loop(0, L, body, init)
        for g in range(groups):
            out_v[pl.ds(g * _LANES, _LANES)] = accs[g]
        pltpu.sync_copy(out_v, out_hbm.at[pl.ds(wid * b_per_w, b_per_w)])

    return gather_sum


def kernel(text, W, b):
    L, B = text.shape
    V = W.shape[1]
    b_per_w = B // _NUM_WORKERS
    tex3 = text.reshape(L, _NUM_WORKERS, b_per_w).transpose(1, 0, 2)
    sums = _make_gather_sum(L, B, V)(W.reshape(V), tex3)
    return sums.reshape(B, 1) + b


# speedup vs baseline: 27.9947x; 1.1470x over previous
import functools

import jax
import jax.numpy as jnp
from jax import lax
from jax.experimental import pallas as pl
from jax.experimental.pallas import tpu as pltpu
from jax.experimental.pallas import tpu_sc as plsc

_NUM_CORES = 2
_NUM_SUBCORES = 16
_NUM_WORKERS = _NUM_CORES * _NUM_SUBCORES
_LANES = 16
_UNROLL = 8


@functools.lru_cache(maxsize=None)
def _make_gather_sum(L, B, V):
    b_per_w = B // _NUM_WORKERS
    groups = b_per_w // _LANES
    mesh = plsc.VectorSubcoreMesh(core_axis_name="c", subcore_axis_name="s")

    @functools.partial(
        pl.kernel,
        mesh=mesh,
        out_type=jax.ShapeDtypeStruct((1, B), jnp.float32),
        scratch_types=[
            pltpu.VMEM((V,), jnp.float32),
            pltpu.VMEM((L, b_per_w), jnp.int32),
            pltpu.VMEM((b_per_w,), jnp.float32),
            pltpu.VMEM((_LANES,), jnp.float32),
            pltpu.SemaphoreType.DMA,
            pltpu.SemaphoreType.DMA,
        ],
        compiler_params=pltpu.CompilerParams(
            needs_layout_passes=False, use_tc_tiling_on_sc=False),
    )
    def gather_sum(w_hbm, tex_hbm, bias_hbm, out_hbm, w_v, tex_v, out_v,
                   bias_v, sem_w, sem_t):
        wid = lax.axis_index("s") * _NUM_CORES + lax.axis_index("c")
        base = wid * b_per_w
        cp_w = pltpu.async_copy(w_hbm, w_v, sem_w)
        cp_t = pltpu.async_copy(tex_hbm.at[:, pl.ds(base, b_per_w)], tex_v,
                                sem_t)
        pltpu.sync_copy(bias_hbm, bias_v.at[pl.ds(0, 1)])
        cp_t.wait()
        cp_w.wait()
        bias = bias_v[...][0]

        def body(i, accs):
            out = list(accs)
            for u in range(_UNROLL):
                l = i * _UNROLL + u
                for g in range(groups):
                    idx = tex_v[l, pl.ds(g * _LANES, _LANES)]
                    vals = plsc.load_gather(w_v, [idx])
                    out[g] = out[g] + vals
            return tuple(out)

        init = tuple(jnp.zeros((_LANES,), jnp.float32) for _ in range(groups))
        accs = lax.fori_loop(0, L // _UNROLL, body, init)
        for l in range((L // _UNROLL) * _UNROLL, L):
            accs = tuple(
                accs[g] + plsc.load_gather(
                    w_v, [tex_v[l, pl.ds(g * _LANES, _LANES)]])
                for g in range(groups)
            )
        for g in range(groups):
            out_v[pl.ds(g * _LANES, _LANES)] = accs[g] + bias
        pltpu.sync_copy(out_v, out_hbm.at[0, pl.ds(base, b_per_w)])

    return gather_sum


def kernel(text, W, b):
    L, B = text.shape
    V = W.shape[1]
    out = _make_gather_sum(L, B, V)(W.reshape(V), text, b)
    return out.reshape(B, 1)
